# baseline (speedup 1.0000x reference)
import functools

import jax
import jax.numpy as jnp
from jax import lax
from jax.experimental import pallas as pl
from jax.experimental.pallas import tpu as pltpu
from jax.experimental.pallas import tpu_sc as plsc

T = 2
V = 64
B = 16384
L = 20

NC = 2
NS = 16
NW = NC * NS
NB = B // NW
RPB = (T + 1) * L
GB = 2
GI = GB * RPB
NG = NB // GB
NBUF = 2
FLUSH_GROUPS = 64
FLUSH_B = FLUSH_GROUPS * GB

AB = 64
NAG = NB // AB


def _sigmoid(x):
    return 1.0 / (1.0 + jnp.exp(-x))


def _qs(q):
    return pl.ds(q * 16, 16)


def _lanesum(x, vi):
    for sh in (8, 4, 2, 1):
        x = x + x.at[vi ^ sh].get(mode="promise_in_bounds")
    return x


def _sc_body(gidx_h, aidx_h, gold_h, act_h, og_h, ag_h, w_h, vb_h,
             feat_h, actp_h, base_h,
             gidx_v, aidx_v, grow_v, arow_v, sg_v, sa_v, wv_v, bv_v,
             feat_s, act_s, abase_v, base_s,
             asem0, asem1, sem0, sem1):
    gsems = (sem0, sem1)
    wid = lax.axis_index("s") * NC + lax.axis_index("c")
    bbase = wid * NB

    pltpu.sync_copy(gidx_h.at[pl.ds(wid * NG, NG)], gidx_v)
    pltpu.sync_copy(aidx_h.at[pl.ds(wid * NAG, NAG)], aidx_v)
    pltpu.sync_copy(og_h, sg_v)
    pltpu.sync_copy(ag_h, sa_v)
    pltpu.sync_copy(w_h, wv_v)
    pltpu.sync_copy(vb_h, bv_v)

    for s in range(T + 1):
        for q in range(4):
            sg_v[s, _qs(q)] = _sigmoid(sg_v[s, _qs(q)])
    for s in range(T):
        for q in range(4):
            sa_v[s, _qs(q)] = _sigmoid(sa_v[s, _qs(q)])

    for buf in range(NBUF):
        pltpu.async_copy(gold_h.at[gidx_v.at[buf]], grow_v.at[buf],
                         gsems[buf])

    b0 = bv_v[_qs(0)]
    vi = lax.iota(jnp.int32, 16)
    ga = [sa_v[0, _qs(q)] for q in range(4)]
    gb = [sa_v[1, _qs(q)] for q in range(4)]
    wa = [wv_v[1, _qs(q)] for q in range(4)]
    wf = [wv_v[0, _qs(q)] for q in range(4)]
    go = [[sg_v[s, _qs(q)] for q in range(4)] for s in range(T + 1)]

    pltpu.async_copy(act_h.at[aidx_v.at[0]], arow_v.at[0], asem0)
    asems = (asem0, asem1)
    for a in range(NAG):
        cur = a & 1
        pltpu.make_async_copy(act_h.at[aidx_v.at[a]], arow_v.at[cur],
                              asems[cur]).wait()
        if a + 1 < NAG:
            pltpu.async_copy(act_h.at[aidx_v.at[a + 1]],
                             arow_v.at[(a + 1) & 1], asems[(a + 1) & 1])

        def abody(j, _, cur=cur, a=a):
            dvec = None
            for q in range(4):
                acc = (arow_v[cur, 2 * j, _qs(q)] * ga[q]
                       + arow_v[cur, 2 * j + 1, _qs(q)] * gb[q])
                act_s[j, _qs(q)] = _sigmoid(acc)
                pw = acc * wa[q]
                dvec = pw if dvec is None else dvec + pw
            val = _lanesum(dvec, vi) + b0
            plsc.store_scatter(abase_v,
                               [jnp.broadcast_to(a * AB + j, (16,))],
                               val, mask=vi == 0)
            return 0

        lax.fori_loop(0, AB, abody, 0)
        pltpu.sync_copy(act_s, actp_h.at[pl.ds(bbase + a * AB, AB)])

    def gbody(i, _):
        for buf in range(NBUF):
            g = i * NBUF + buf
            pltpu.make_async_copy(gold_h.at[gidx_v.at[g]],
                                  grow_v.at[buf], gsems[buf]).wait()
            av = abase_v[pl.ds(GB * g, 16)]
            bvals = []
            for b2 in range(GB):
                r0 = b2 * RPB
                facc = [None] * 4
                for s in range(T + 1):
                    base = r0 + s * L
                    ssum = [None] * 4
                    for l in range(L):
                        for q in range(4):
                            v = grow_v[buf, base + l, _qs(q)]
                            ssum[q] = v if ssum[q] is None else ssum[q] + v
                    for q in range(4):
                        gs = ssum[q] * go[s][q]
                        facc[q] = gs if facc[q] is None else facc[q] + gs
                rl = lax.bitwise_and(GB * g + b2, FLUSH_B - 1)
                dvec = None
                for q in range(4):
                    feat_s[rl, _qs(q)] = _sigmoid(facc[q])
                    pw = facc[q] * wf[q]
                    dvec = pw if dvec is None else dvec + pw
                bvals.append(_lanesum(dvec, vi) + av[b2])
            rl0 = lax.bitwise_and(GB * g, FLUSH_B - 1)
            bv2 = jnp.where(vi == 0, bvals[0], bvals[1])
            plsc.store_scatter(base_s, [rl0 + vi], bv2, mask=vi < GB)
            gn = g + NBUF

            @pl.when(gn < NG)
            def _():
                pltpu.async_copy(gold_h.at[gidx_v.at[gn]],
                                 grow_v.at[buf], gsems[buf])

        @pl.when(lax.rem(i + 1, FLUSH_GROUPS // NBUF) == 0)
        def _():
            blk = i // (FLUSH_GROUPS // NBUF)
            off = bbase + blk * FLUSH_B
            pltpu.sync_copy(feat_s, feat_h.at[pl.ds(off, FLUSH_B)])
            pltpu.sync_copy(base_s, base_h.at[pl.ds(off, FLUSH_B)])
        return 0

    lax.fori_loop(0, NG // NBUF, gbody, 0)


@jax.jit
def _run(gidx2d, aidx2d, gold_table, act_table, og, ag, w2, vb16):
    mesh = plsc.VectorSubcoreMesh(core_axis_name="c", subcore_axis_name="s",
                                  num_cores=NC, num_subcores=NS)
    f = pl.kernel(
        _sc_body,
        out_type=[
            jax.ShapeDtypeStruct((B, V), jnp.float32),
            jax.ShapeDtypeStruct((B, V), jnp.float32),
            jax.ShapeDtypeStruct((B,), jnp.float32),
        ],
        mesh=mesh,
        compiler_params=pltpu.CompilerParams(needs_layout_passes=False,
                                             use_tc_tiling_on_sc=False),
        scratch_types=[
            pltpu.VMEM((NG, GI), jnp.int32),
            pltpu.VMEM((NAG, 2 * AB), jnp.int32),
            pltpu.VMEM((NBUF, GI, V), jnp.float32),
            pltpu.VMEM((2, 2 * AB, V), jnp.float32),
            pltpu.VMEM((T + 1, V), jnp.float32),
            pltpu.VMEM((T, V), jnp.float32),
            pltpu.VMEM((2, V), jnp.float32),
            pltpu.VMEM((16,), jnp.float32),
            pltpu.VMEM((FLUSH_B, V), jnp.float32),
            pltpu.VMEM((AB, V), jnp.float32),
            pltpu.VMEM((NB + 16,), jnp.float32),
            pltpu.VMEM((FLUSH_B,), jnp.float32),
            pltpu.SemaphoreType.DMA,
            pltpu.SemaphoreType.DMA,
            pltpu.SemaphoreType.DMA,
            pltpu.SemaphoreType.DMA,
        ],
    )
    return f(gidx2d, aidx2d, gold_table, act_table, og, ag, w2, vb16)


def kernel(goldstandard, actions, gold_table, act_table, obs_gates,
           act_gates, value_W, value_b):
    d2 = jnp.concatenate([gold_table[0::2, :], gold_table[1::2, :]], axis=1)
    gt3 = d2.reshape(-1, V)
    gidx2d = goldstandard.astype(jnp.int32).reshape(B // GB, GI)
    aidx2d = actions.astype(jnp.int32).reshape(B // AB, 2 * AB)
    og = obs_gates.reshape(T + 1, V)
    ag = act_gates.reshape(T, V)
    w2 = value_W.reshape(2, V)
    vb16 = jnp.broadcast_to(value_b, (16,))
    feat_p, act_p, base = _run(gidx2d, aidx2d, gt3, act_table,
                               og, ag, w2, vb16)
    return jnp.concatenate([feat_p, act_p, base[:, None]], axis=1)


# speedup vs baseline: 9.3924x; 9.3924x over previous
import functools

import jax
import jax.numpy as jnp
from jax import lax
from jax.experimental import pallas as pl
from jax.experimental.pallas import tpu as pltpu
from jax.experimental.pallas import tpu_sc as plsc

T = 2
V = 64
B = 16384
L = 20

NC = 2
NS = 16
NW = NC * NS
NB = B // NW
RPB = (T + 1) * L
GB = 2
GI = GB * RPB
NG = NB // GB
NBUF = 4
FLUSH_GROUPS = 64
FLUSH_B = FLUSH_GROUPS * GB

AB = 64
NAG = NB // AB


def _sigmoid(x):
    return 1.0 / (1.0 + jnp.exp(-x))


def _qs(q):
    return pl.ds(q * 16, 16)


def _lanesum(x, vi):
    for sh in (8, 4, 2, 1):
        x = x + x.at[vi ^ sh].get(mode="promise_in_bounds")
    return x


def _sc_body(gidx_h, aidx_h, gold_h, act_h, og_h, ag_h, w_h, vb_h,
             feat_h, actp_h, base_h,
             gidx_v, aidx_v, grow_v, arow_v, sg_v, sa_v, wv_v, bv_v,
             feat_s, act_s, abase_v, base_s,
             asem0, asem1, sem0, sem1, sem2, sem3):
    gsems = (sem0, sem1, sem2, sem3)
    wid = lax.axis_index("s") * NC + lax.axis_index("c")
    bbase = wid * NB

    pltpu.sync_copy(gidx_h.at[pl.ds(wid * NG, NG)], gidx_v)
    pltpu.sync_copy(aidx_h.at[pl.ds(wid * NAG, NAG)], aidx_v)
    pltpu.sync_copy(og_h, sg_v)
    pltpu.sync_copy(ag_h, sa_v)
    pltpu.sync_copy(w_h, wv_v)
    pltpu.sync_copy(vb_h, bv_v)

    for s in range(T + 1):
        for q in range(4):
            sg_v[s, _qs(q)] = _sigmoid(sg_v[s, _qs(q)])
    for s in range(T):
        for q in range(4):
            sa_v[s, _qs(q)] = _sigmoid(sa_v[s, _qs(q)])

    for buf in range(NBUF):
        pltpu.async_copy(gold_h.at[gidx_v.at[buf]], grow_v.at[buf],
                         gsems[buf])

    b0 = bv_v[_qs(0)]
    vi = lax.iota(jnp.int32, 16)
    ga = [sa_v[0, _qs(q)] for q in range(4)]
    gb = [sa_v[1, _qs(q)] for q in range(4)]
    wa = [wv_v[1, _qs(q)] for q in range(4)]
    wf = [wv_v[0, _qs(q)] for q in range(4)]
    go = [[sg_v[s, _qs(q)] for q in range(4)] for s in range(T + 1)]

    pltpu.async_copy(act_h.at[aidx_v.at[0]], arow_v.at[0], asem0)
    asems = (asem0, asem1)
    for a in range(NAG):
        cur = a & 1
        pltpu.make_async_copy(act_h.at[aidx_v.at[a]], arow_v.at[cur],
                              asems[cur]).wait()
        if a + 1 < NAG:
            pltpu.async_copy(act_h.at[aidx_v.at[a + 1]],
                             arow_v.at[(a + 1) & 1], asems[(a + 1) & 1])

        def abody(j, _, cur=cur, a=a):
            dvec = None
            for q in range(4):
                acc = (arow_v[cur, 2 * j, _qs(q)] * ga[q]
                       + arow_v[cur, 2 * j + 1, _qs(q)] * gb[q])
                act_s[j, _qs(q)] = _sigmoid(acc)
                pw = acc * wa[q]
                dvec = pw if dvec is None else dvec + pw
            val = _lanesum(dvec, vi) + b0
            plsc.store_scatter(abase_v,
                               [jnp.broadcast_to(a * AB + j, (16,))],
                               val, mask=vi == 0)
            return 0

        lax.fori_loop(0, AB, abody, 0)
        pltpu.sync_copy(act_s, actp_h.at[pl.ds(bbase + a * AB, AB)])

    def gbody(i, _):
        for buf in range(NBUF):
            g = i * NBUF + buf
            pltpu.make_async_copy(gold_h.at[gidx_v.at[g]],
                                  grow_v.at[buf], gsems[buf]).wait()
            av = abase_v[pl.ds(GB * g, 16)]
            bvals = []
            for b2 in range(GB):
                r0 = b2 * RPB
                facc = [None] * 4
                for s in range(T + 1):
                    def lbody(l, c, _r0=r0, _s=s):
                        r = _r0 + _s * L + l * 4
                        c = list(c)
                        for q in range(4):
                            c[q] = (c[q]
                                    + grow_v[buf, r, _qs(q)]
                                    + grow_v[buf, r + 1, _qs(q)]
                                    + grow_v[buf, r + 2, _qs(q)]
                                    + grow_v[buf, r + 3, _qs(q)])
                        return tuple(c)
                    z = jnp.zeros((16,), jnp.float32)
                    ssum = lax.fori_loop(0, L // 4, lbody, (z, z, z, z))
                    for q in range(4):
                        gs = ssum[q] * go[s][q]
                        facc[q] = gs if facc[q] is None else facc[q] + gs
                rl = lax.bitwise_and(GB * g + b2, FLUSH_B - 1)
                dvec = None
                for q in range(4):
                    feat_s[rl, _qs(q)] = _sigmoid(facc[q])
                    pw = facc[q] * wf[q]
                    dvec = pw if dvec is None else dvec + pw
                bvals.append(_lanesum(dvec, vi) + av[b2])
            rl0 = lax.bitwise_and(GB * g, FLUSH_B - 1)
            bv2 = jnp.where(vi == 0, bvals[0], bvals[1])
            plsc.store_scatter(base_s, [rl0 + vi], bv2, mask=vi < GB)
            gn = g + NBUF

            @pl.when(gn < NG)
            def _():
                pltpu.async_copy(gold_h.at[gidx_v.at[gn]],
                                 grow_v.at[buf], gsems[buf])

        @pl.when(lax.rem(i + 1, FLUSH_GROUPS // NBUF) == 0)
        def _():
            blk = i // (FLUSH_GROUPS // NBUF)
            off = bbase + blk * FLUSH_B
            pltpu.sync_copy(feat_s, feat_h.at[pl.ds(off, FLUSH_B)])
            pltpu.sync_copy(base_s, base_h.at[pl.ds(off, FLUSH_B)])
        return 0

    lax.fori_loop(0, NG // NBUF, gbody, 0)


@jax.jit
def _run(gidx2d, aidx2d, gold_table, act_table, og, ag, w2, vb16):
    mesh = plsc.VectorSubcoreMesh(core_axis_name="c", subcore_axis_name="s",
                                  num_cores=NC, num_subcores=NS)
    f = pl.kernel(
        _sc_body,
        out_type=[
            jax.ShapeDtypeStruct((B, V), jnp.float32),
            jax.ShapeDtypeStruct((B, V), jnp.float32),
            jax.ShapeDtypeStruct((B,), jnp.float32),
        ],
        mesh=mesh,
        compiler_params=pltpu.CompilerParams(needs_layout_passes=False,
                                             use_tc_tiling_on_sc=False),
        scratch_types=[
            pltpu.VMEM((NG, GI), jnp.int32),
            pltpu.VMEM((NAG, 2 * AB), jnp.int32),
            pltpu.VMEM((NBUF, GI, V), jnp.float32),
            pltpu.VMEM((2, 2 * AB, V), jnp.float32),
            pltpu.VMEM((T + 1, V), jnp.float32),
            pltpu.VMEM((T, V), jnp.float32),
            pltpu.VMEM((2, V), jnp.float32),
            pltpu.VMEM((16,), jnp.float32),
            pltpu.VMEM((FLUSH_B, V), jnp.float32),
            pltpu.VMEM((AB, V), jnp.float32),
            pltpu.VMEM((NB + 16,), jnp.float32),
            pltpu.VMEM((FLUSH_B,), jnp.float32),
            pltpu.SemaphoreType.DMA,
            pltpu.SemaphoreType.DMA,
            pltpu.SemaphoreType.DMA,
            pltpu.SemaphoreType.DMA,
            pltpu.SemaphoreType.DMA,
            pltpu.SemaphoreType.DMA,
        ],
    )
    return f(gidx2d, aidx2d, gold_table, act_table, og, ag, w2, vb16)


def kernel(goldstandard, actions, gold_table, act_table, obs_gates,
           act_gates, value_W, value_b):
    gidx2d = goldstandard.astype(jnp.int32).reshape(B // GB, GI)
    aidx2d = actions.astype(jnp.int32).reshape(B // AB, 2 * AB)
    og = obs_gates.reshape(T + 1, V)
    ag = act_gates.reshape(T, V)
    w2 = value_W.reshape(2, V)
    vb16 = jnp.broadcast_to(value_b, (16,))
    feat_p, act_p, base = _run(gidx2d, aidx2d, gold_table, act_table,
                               og, ag, w2, vb16)
    return jnp.concatenate([feat_p, act_p, base[:, None]], axis=1)


# speedup vs baseline: 9.4237x; 1.0033x over previous
import jax
import jax.numpy as jnp
from jax import lax
from jax.experimental import pallas as pl
from jax.experimental.pallas import tpu as pltpu
from jax.experimental.pallas import tpu_sc as plsc

T = 2
V = 64
B = 16384
L = 20

NC = 2
NS = 16
NW = NC * NS
NB = B // NW
RPB = (T + 1) * L
GB = 2
GI = GB * RPB
NG = NB // GB
NBUF = 4
FLUSH_GROUPS = 64
FLUSH_B = FLUSH_GROUPS * GB

AB = 64
NAG = NB // AB


def _sigmoid(x):
    return 1.0 / (1.0 + jnp.exp(-x))


def _qs(q):
    return pl.ds(q * 16, 16)


def _lanesum(x, vi):
    for sh in (8, 4, 2, 1):
        x = x + x.at[vi ^ sh].get(mode="promise_in_bounds")
    return x


def _sc_body(gidx_h, aidx_h, gold_h, act_h, og_h, ag_h, w_h, vb_h,
             feat_h, actp_h, base_h,
             gidx_v, aidx_v, grow_v, arow_v, sg_v, sa_v, wv_v, bv_v,
             feat_s, act_s, abase_v, base_s,
             asem0, asem1, sem0, sem1, sem2, sem3):
    gsems = (sem0, sem1, sem2, sem3)
    wid = lax.axis_index("s") * NC + lax.axis_index("c")
    bbase = wid * NB

    pltpu.sync_copy(gidx_h.at[pl.ds(wid * NG, NG)], gidx_v)
    pltpu.sync_copy(aidx_h.at[pl.ds(wid * NAG, NAG)], aidx_v)
    pltpu.sync_copy(og_h, sg_v)
    pltpu.sync_copy(ag_h, sa_v)
    pltpu.sync_copy(w_h, wv_v)
    pltpu.sync_copy(vb_h, bv_v)

    for s in range(T + 1):
        for q in range(4):
            sg_v[s, _qs(q)] = _sigmoid(sg_v[s, _qs(q)])
    for s in range(T):
        for q in range(4):
            sa_v[s, _qs(q)] = _sigmoid(sa_v[s, _qs(q)])

    for buf in range(NBUF):
        pltpu.async_copy(gold_h.at[gidx_v.at[buf]], grow_v.at[buf],
                         gsems[buf])

    b0 = bv_v[_qs(0)]
    vi = lax.iota(jnp.int32, 16)
    ga = [sa_v[0, _qs(q)] for q in range(4)]
    gb = [sa_v[1, _qs(q)] for q in range(4)]
    wa = [wv_v[1, _qs(q)] for q in range(4)]
    wf = [wv_v[0, _qs(q)] for q in range(4)]
    go = [[sg_v[s, _qs(q)] for q in range(4)] for s in range(T + 1)]

    pltpu.async_copy(act_h.at[aidx_v.at[0]], arow_v.at[0], asem0)
    asems = (asem0, asem1)
    for a in range(NAG):
        cur = a & 1
        pltpu.make_async_copy(act_h.at[aidx_v.at[a]], arow_v.at[cur],
                              asems[cur]).wait()
        if a + 1 < NAG:
            pltpu.async_copy(act_h.at[aidx_v.at[a + 1]],
                             arow_v.at[(a + 1) & 1], asems[(a + 1) & 1])

        def abody(j, _, cur=cur, a=a):
            dvec = None
            for q in range(4):
                acc = (arow_v[cur, 2 * j, _qs(q)] * ga[q]
                       + arow_v[cur, 2 * j + 1, _qs(q)] * gb[q])
                act_s[j, _qs(q)] = _sigmoid(acc)
                pw = acc * wa[q]
                dvec = pw if dvec is None else dvec + pw
            val = _lanesum(dvec, vi) + b0
            plsc.store_scatter(abase_v,
                               [jnp.broadcast_to(a * AB + j, (16,))],
                               val, mask=vi == 0)
            return 0

        lax.fori_loop(0, AB, abody, 0)
        pltpu.sync_copy(act_s, actp_h.at[pl.ds(bbase + a * AB, AB)])

    def gbody(i, _):
        for buf in range(NBUF):
            g = i * NBUF + buf
            pltpu.make_async_copy(gold_h.at[gidx_v.at[g]],
                                  grow_v.at[buf], gsems[buf]).wait()
            av = abase_v[pl.ds(GB * g, 16)]
            bvals = []
            for b2 in range(GB):
                r0 = b2 * RPB
                facc = [None] * 4
                for s in range(T + 1):
                    def lbody(l, c, _r0=r0, _s=s):
                        r = _r0 + _s * L + l * 4
                        c = list(c)
                        for q in range(4):
                            c[q] = (c[q]
                                    + grow_v[buf, r, _qs(q)]
                                    + grow_v[buf, r + 1, _qs(q)]
                                    + grow_v[buf, r + 2, _qs(q)]
                                    + grow_v[buf, r + 3, _qs(q)])
                        return tuple(c)
                    z = jnp.zeros((16,), jnp.float32)
                    ssum = lax.fori_loop(0, L // 4, lbody, (z, z, z, z))
                    for q in range(4):
                        gs = ssum[q] * go[s][q]
                        facc[q] = gs if facc[q] is None else facc[q] + gs
                rl = lax.bitwise_and(GB * g + b2, FLUSH_B - 1)
                dvec = None
                for q in range(4):
                    feat_s[rl, _qs(q)] = _sigmoid(facc[q])
                    pw = facc[q] * wf[q]
                    dvec = pw if dvec is None else dvec + pw
                bvals.append(_lanesum(dvec, vi) + av[b2])
            rl0 = lax.bitwise_and(GB * g, FLUSH_B - 1)
            bv2 = jnp.where(vi == 0, bvals[0], bvals[1])
            plsc.store_scatter(base_s, [rl0 + vi], bv2, mask=vi < GB)
            gn = g + NBUF

            @pl.when(gn < NG)
            def _():
                pltpu.async_copy(gold_h.at[gidx_v.at[gn]],
                                 grow_v.at[buf], gsems[buf])

        @pl.when(lax.rem(i + 1, FLUSH_GROUPS // NBUF) == 0)
        def _():
            blk = i // (FLUSH_GROUPS // NBUF)
            off = bbase + blk * FLUSH_B
            pltpu.sync_copy(feat_s, feat_h.at[pl.ds(off, FLUSH_B)])
            pltpu.sync_copy(base_s, base_h.at[pl.ds(off, FLUSH_B)])
        return 0

    lax.fori_loop(0, NG // NBUF, gbody, 0)


@jax.jit
def _run(gidx2d, aidx2d, gold_table, act_table, og, ag, w2, vb16):
    mesh = plsc.VectorSubcoreMesh(core_axis_name="c", subcore_axis_name="s",
                                  num_cores=NC, num_subcores=NS)
    f = pl.kernel(
        _sc_body,
        out_type=[
            jax.ShapeDtypeStruct((B, V), jnp.float32),
            jax.ShapeDtypeStruct((B, V), jnp.float32),
            jax.ShapeDtypeStruct((B,), jnp.float32),
        ],
        mesh=mesh,
        compiler_params=pltpu.CompilerParams(needs_layout_passes=False,
                                             use_tc_tiling_on_sc=False),
        scratch_types=[
            pltpu.VMEM((NG, GI), jnp.int32),
            pltpu.VMEM((NAG, 2 * AB), jnp.int32),
            pltpu.VMEM((NBUF, GI, V), jnp.float32),
            pltpu.VMEM((2, 2 * AB, V), jnp.float32),
            pltpu.VMEM((T + 1, V), jnp.float32),
            pltpu.VMEM((T, V), jnp.float32),
            pltpu.VMEM((2, V), jnp.float32),
            pltpu.VMEM((16,), jnp.float32),
            pltpu.VMEM((FLUSH_B, V), jnp.float32),
            pltpu.VMEM((AB, V), jnp.float32),
            pltpu.VMEM((NB + 16,), jnp.float32),
            pltpu.VMEM((FLUSH_B,), jnp.float32),
            pltpu.SemaphoreType.DMA,
            pltpu.SemaphoreType.DMA,
            pltpu.SemaphoreType.DMA,
            pltpu.SemaphoreType.DMA,
            pltpu.SemaphoreType.DMA,
            pltpu.SemaphoreType.DMA,
        ],
    )
    return f(gidx2d, aidx2d, gold_table, act_table, og, ag, w2, vb16)


def kernel(goldstandard, actions, gold_table, act_table, obs_gates,
           act_gates, value_W, value_b):
    gidx2d = goldstandard.astype(jnp.int32).reshape(B // GB, GI)
    aidx2d = actions.astype(jnp.int32).reshape(B // AB, 2 * AB)
    og = obs_gates.reshape(T + 1, V)
    ag = act_gates.reshape(T, V)
    w2 = value_W.reshape(2, V)
    vb16 = jnp.broadcast_to(value_b, (16,))
    feat_p, act_p, base = _run(gidx2d, aidx2d, gold_table, act_table,
                               og, ag, w2, vb16)
    return jnp.concatenate([feat_p, act_p, base[:, None]], axis=1)
